# baseline (speedup 1.0000x reference)
import jax
import jax.numpy as jnp
from jax.experimental import pallas as pl
from jax.experimental.pallas import tpu as pltpu

_P = 2
_LEVELS = 3


def _sppf_kernel(x_ref, w1_ref, s1_ref, b1_ref, w2_ref, s2_ref, b2_ref,
                 o_ref, pad_ref, row_ref, acc_ref):
    B = x_ref.shape[0]
    C = w1_ref.shape[1]
    H = pad_ref.shape[0] - 2 * _P
    W = pad_ref.shape[1] - 2 * _P
    HW = H * W

    pad_ref[...] = jnp.full(pad_ref.shape, -jnp.inf, jnp.bfloat16)

    for b in range(B):
        with jax.named_scope("cv1"):
            xb = x_ref[b].astype(jnp.bfloat16)
            y = jax.lax.dot_general(xb, w1_ref[...], (((0,), (0,)), ((), ())),
                                    preferred_element_type=jnp.float32)
        with jax.named_scope("bn1"):
            y = y * s1_ref[...] + b1_ref[...]
            y = y * jax.nn.sigmoid(y)
            cur = y.astype(jnp.bfloat16)

        with jax.named_scope("mm_id"):
            acc_ref[...] = jnp.dot(cur, w2_ref[0:C, :],
                                   preferred_element_type=jnp.float32)

        for level in range(_LEVELS):
            with jax.named_scope(f"pool{level}"):
                pad_ref[_P:_P + H, _P:_P + W, :] = cur.reshape(H, W, C)
                m = pad_ref[0:H, :, :]
                for d in range(1, 2 * _P + 1):
                    m = jnp.maximum(m, pad_ref[d:d + H, :, :])
                row_ref[...] = m
                m = row_ref[:, 0:W, :]
                for d in range(1, 2 * _P + 1):
                    m = jnp.maximum(m, row_ref[:, d:d + W, :])
            with jax.named_scope(f"reshape{level}"):
                cur = m.reshape(HW, C)
            with jax.named_scope(f"mm{level}"):
                acc_ref[...] += jnp.dot(
                    cur, w2_ref[(level + 1) * C:(level + 2) * C, :],
                    preferred_element_type=jnp.float32)

        with jax.named_scope("bn2"):
            z = acc_ref[...] * s2_ref[...] + b2_ref[...]
            z = z * jax.nn.sigmoid(z)
        with jax.named_scope("outT"):
            o_ref[b] = z.T


def kernel(x, w1, scale1, bias1, w2, scale2, bias2):
    n, c1, h, w = x.shape
    c = w1.shape[1]
    c2 = w2.shape[1]
    hw = h * w

    x3 = x.reshape(n, c1, hw)
    w1b = w1.astype(jnp.bfloat16)
    w2b = w2.astype(jnp.bfloat16)

    bb = 4

    flops = 2 * n * hw * c1 * c + 2 * n * hw * (4 * c) * c2
    bytes_accessed = 4 * (n * c1 * hw + n * c2 * hw + 2 * c + 2 * c2) \
        + 2 * (c1 * c + 4 * c * c2)

    out = pl.pallas_call(
        _sppf_kernel,
        out_shape=jax.ShapeDtypeStruct((n, c2, hw), jnp.float32),
        grid=(n // bb,),
        in_specs=[
            pl.BlockSpec((bb, c1, hw), lambda i: (i, 0, 0)),
            pl.BlockSpec((c1, c), lambda i: (0, 0)),
            pl.BlockSpec((1, c), lambda i: (0, 0)),
            pl.BlockSpec((1, c), lambda i: (0, 0)),
            pl.BlockSpec((4 * c, c2), lambda i: (0, 0)),
            pl.BlockSpec((1, c2), lambda i: (0, 0)),
            pl.BlockSpec((1, c2), lambda i: (0, 0)),
        ],
        out_specs=pl.BlockSpec((bb, c2, hw), lambda i: (i, 0, 0)),
        scratch_shapes=[
            pltpu.VMEM((h + 2 * _P, w + 2 * _P, c), jnp.bfloat16),
            pltpu.VMEM((h, w + 2 * _P, c), jnp.bfloat16),
            pltpu.VMEM((hw, c2), jnp.float32),
        ],
        compiler_params=pltpu.CompilerParams(
            dimension_semantics=("parallel",)),
        cost_estimate=pl.CostEstimate(
            flops=flops, transcendentals=n * hw * (c + c2),
            bytes_accessed=bytes_accessed),
    )(x3, w1b, scale1, bias1, w2b, scale2, bias2)

    return out.reshape(n, c2, h, w)


# speedup vs baseline: 1.0082x; 1.0082x over previous
import jax
import jax.numpy as jnp
from jax.experimental import pallas as pl
from jax.experimental.pallas import tpu as pltpu

_P = 2
_LEVELS = 3


def _sppf_kernel(x_ref, w1_ref, s1_ref, b1_ref, w2_ref, s2_ref, b2_ref,
                 o_ref, pad_ref, row_ref, acc_ref):
    B = x_ref.shape[0]
    C = w1_ref.shape[1]
    H = pad_ref.shape[0] - 2 * _P
    W = pad_ref.shape[1] - 2 * _P
    HW = H * W

    pad_ref[...] = jnp.full(pad_ref.shape, -jnp.inf, jnp.bfloat16)

    for b in range(B):
        with jax.named_scope("cv1"):
            xb = x_ref[b].astype(jnp.bfloat16)
            y = jax.lax.dot_general(xb, w1_ref[...], (((0,), (0,)), ((), ())),
                                    preferred_element_type=jnp.float32)
        with jax.named_scope("bn1"):
            y = y * s1_ref[...] + b1_ref[...]
            y = y * jax.nn.sigmoid(y)
            cur = y.astype(jnp.bfloat16)

        with jax.named_scope("cat_id"):
            acc_ref[:, 0:C] = cur

        for level in range(_LEVELS):
            with jax.named_scope(f"pool{level}"):
                pad_ref[_P:_P + H, _P:_P + W, :] = cur.reshape(H, W, C)
                m = pad_ref[0:H, :, :]
                for d in range(1, 2 * _P + 1):
                    m = jnp.maximum(m, pad_ref[d:d + H, :, :])
                row_ref[...] = m
                m = row_ref[:, 0:W, :]
                for d in range(1, 2 * _P + 1):
                    m = jnp.maximum(m, row_ref[:, d:d + W, :])
            with jax.named_scope(f"cat{level}"):
                cur = m.reshape(HW, C)
                acc_ref[:, (level + 1) * C:(level + 2) * C] = cur

        with jax.named_scope("mm2"):
            z = jnp.dot(acc_ref[...], w2_ref[...],
                        preferred_element_type=jnp.float32)
        with jax.named_scope("bn2"):
            z = z * s2_ref[...] + b2_ref[...]
            z = z * jax.nn.sigmoid(z)
        with jax.named_scope("outT"):
            o_ref[b] = z.T


def kernel(x, w1, scale1, bias1, w2, scale2, bias2):
    n, c1, h, w = x.shape
    c = w1.shape[1]
    c2 = w2.shape[1]
    hw = h * w

    x3 = x.reshape(n, c1, hw)
    w1b = w1.astype(jnp.bfloat16)
    w2b = w2.astype(jnp.bfloat16)

    bb = 4

    flops = 2 * n * hw * c1 * c + 2 * n * hw * (4 * c) * c2
    bytes_accessed = 4 * (n * c1 * hw + n * c2 * hw + 2 * c + 2 * c2) \
        + 2 * (c1 * c + 4 * c * c2)

    out = pl.pallas_call(
        _sppf_kernel,
        out_shape=jax.ShapeDtypeStruct((n, c2, hw), jnp.float32),
        grid=(n // bb,),
        in_specs=[
            pl.BlockSpec((bb, c1, hw), lambda i: (i, 0, 0)),
            pl.BlockSpec((c1, c), lambda i: (0, 0)),
            pl.BlockSpec((1, c), lambda i: (0, 0)),
            pl.BlockSpec((1, c), lambda i: (0, 0)),
            pl.BlockSpec((4 * c, c2), lambda i: (0, 0)),
            pl.BlockSpec((1, c2), lambda i: (0, 0)),
            pl.BlockSpec((1, c2), lambda i: (0, 0)),
        ],
        out_specs=pl.BlockSpec((bb, c2, hw), lambda i: (i, 0, 0)),
        scratch_shapes=[
            pltpu.VMEM((h + 2 * _P, w + 2 * _P, c), jnp.bfloat16),
            pltpu.VMEM((h, w + 2 * _P, c), jnp.bfloat16),
            pltpu.VMEM((hw, 4 * c), jnp.bfloat16),
        ],
        compiler_params=pltpu.CompilerParams(
            dimension_semantics=("parallel",)),
        cost_estimate=pl.CostEstimate(
            flops=flops, transcendentals=n * hw * (c + c2),
            bytes_accessed=bytes_accessed),
    )(x3, w1b, scale1, bias1, w2b, scale2, bias2)

    return out.reshape(n, c2, h, w)
